# baseline (speedup 1.0000x reference)
import jax
import jax.numpy as jnp
from jax.experimental import pallas as pl
from jax.experimental.pallas import tpu as pltpu

_B = 16
_CLS = 16
_HS = (64, 32, 16, 8, 4)
_HH = tuple(h * h for h in _HS)
_NPB = sum(_HH)
_N = _B * _NPB
_K = max(int(_N * 0.01), 2)
_OFF = (0, 4096, 5120, 5376, 5440)
_NLVL = 5
_ONE_BITS = 0x3F800001
_LANE0 = (0, 64, 96, 112, 120)
_DEAD = 1 << 30
_GB = 1


def _smooth_l1(x, y):
    d = jnp.abs(x - y)
    return jnp.where(d < 1.0, 0.5 * d * d, d - 0.5)


def _rect(lvl):
    h = _HS[lvl]
    return slice(0, h), slice(_LANE0[lvl], _LANE0[lvl] + h)


def _pattern():
    row = jax.lax.broadcasted_iota(jnp.int32, (64, 128), 0)
    lane = jax.lax.broadcasted_iota(jnp.int32, (64, 128), 1)
    p = jnp.full((64, 128), _DEAD, jnp.int32)
    for l in range(_NLVL - 1, -1, -1):
        h = _HS[l]
        inside = (lane >= _LANE0[l]) & (lane < _LANE0[l] + h) & (row < h)
        p = jnp.where(inside, _OFF[l] + row * h + (lane - _LANE0[l]), p)
    return p


def _body(*refs):
    t_cls = refs[0:5]
    t_bbox = refs[5:10]
    t_angle = refs[10:15]
    t_ctr = refs[15:20]
    s_cls = refs[20:25]
    s_bbox = refs[25:30]
    s_angle = refs[30:35]
    s_ctr = refs[35:40]
    out = refs[40]
    f_s, f_dl, f_bb, f_ct, f_ix = refs[41:46]
    ts3, ss3, bt6, bs6 = refs[46:50]
    f_p, acc = refs[50:52]

    g = pl.program_id(0)

    @pl.when(g == 0)
    def _init():
        acc[0] = jnp.float32(0.0)
        f_p[...] = _pattern()
        ts3[...] = jnp.zeros(ts3.shape, jnp.float32)
        ss3[...] = jnp.zeros(ss3.shape, jnp.float32)
        bt6[...] = jnp.zeros(bt6.shape, jnp.float32)
        bs6[...] = jnp.zeros(bs6.shape, jnp.float32)

    valid = f_p[...] < _DEAD
    base_total = jnp.float32(0.0)
    for bb in range(_GB):
        b = g * _GB + bb
        blk = pl.ds(b * 64, 64)

        for l in range(_NLVL):
            r, c = _rect(l)
            ts3[:, r, c] = t_cls[l][bb]
            ss3[:, r, c] = s_cls[l][bb]
            bt6[0:4, r, c] = t_bbox[l][bb]
            bs6[0:4, r, c] = s_bbox[l][bb]
            bt6[4:5, r, c] = t_angle[l][bb]
            bs6[4:5, r, c] = s_angle[l][bb]
            bt6[5:6, r, c] = t_ctr[l][bb]
            bs6[5:6, r, c] = s_ctr[l][bb]

        x = ss3[...]
        ts = jax.nn.sigmoid(ts3[...])
        ps = jax.nn.sigmoid(x)
        s = jnp.max(ts, axis=0)
        f_s[blk] = jnp.where(valid, s, -1.0)
        f_ix[blk] = f_p[...] + b * _NPB

        l1p = jnp.log1p(-ps)
        base_total += jnp.sum(
            jnp.where(valid[None], l1p * jnp.square(ps), 0.0))
        d = ts - ps
        delta = ts * (x * jnp.square(d) + l1p * (d - ps))
        f_dl[blk] = -jnp.sum(delta, axis=0)

        bt = bt6[...]
        bs = bs6[...]
        d5 = jnp.sum(_smooth_l1(bs[0:5], bt[0:5]), axis=0)
        tc_ = jax.nn.sigmoid(bt[5])
        sc_ = jax.nn.sigmoid(bs[5])
        f_bb[blk] = d5 * tc_
        pcc = jnp.clip(sc_, 1e-12, 1.0 - 1e-12)
        f_ct[blk] = -(tc_ * jnp.log(pcc) + (1.0 - tc_) * jnp.log1p(-pcc))

    acc[0] += -base_total

    @pl.when(g == _B // _GB - 1)
    def _finish():
        _tail(out, f_s, f_dl, f_bb, f_ct, f_ix, acc)


def _tail(out, f_s, f_dl, f_bb, f_ct, f_ix, acc):
    base_total = acc[0]

    def bs_val(i, carry):
        lo, hi = carry
        mid = (lo + hi) // 2
        tf = jax.lax.bitcast_convert_type(mid, jnp.float32)
        big = jnp.sum((f_s[...] >= tf).astype(jnp.int32)) >= _K
        return (jnp.where(big, mid, lo), jnp.where(big, hi, mid))

    lo, _ = jax.lax.fori_loop(
        0, 31, bs_val, (jnp.int32(0), jnp.int32(_ONE_BITS)))
    T = jax.lax.bitcast_convert_type(lo, jnp.float32)

    s = f_s[...]
    ix = f_ix[...]
    c_gt = jnp.sum((s > T).astype(jnp.int32))
    m = _K - c_gt

    def bs_idx(i, carry):
        lo2, hi2 = carry
        mid = (lo2 + hi2) // 2
        c = jnp.sum(((f_s[...] == T) & (f_ix[...] <= mid)).astype(jnp.int32))
        ok = c >= m
        return (jnp.where(ok, lo2, mid), jnp.where(ok, mid, hi2))

    _, I = jax.lax.fori_loop(
        0, 17, bs_idx, (jnp.int32(-1), jnp.int32(_N - 1)))

    mask = (s > T) | ((s == T) & (ix <= I))
    cls_sum = base_total + jnp.sum(jnp.where(mask, f_dl[...], 0.0))
    bbox_sum = jnp.sum(jnp.where(mask, f_bb[...], 0.0))
    ctr_sum = jnp.sum(jnp.where(mask, f_ct[...], 0.0))
    fg_num = jnp.sum(jnp.where(s > T, s, 0.0)) + m.astype(jnp.float32) * T
    s_sum = jnp.sum(jnp.maximum(s, 0.0))

    out[0] = cls_sum / fg_num
    out[1] = bbox_sum / jnp.float32(_K * 5)
    out[2] = ctr_sum / jnp.float32(_K)
    out[3] = s_sum / jnp.float32(_N)


def kernel(t_cls_0, t_bbox_0, t_angle_0, t_ctr_0,
           t_cls_1, t_bbox_1, t_angle_1, t_ctr_1,
           t_cls_2, t_bbox_2, t_angle_2, t_ctr_2,
           t_cls_3, t_bbox_3, t_angle_3, t_ctr_3,
           t_cls_4, t_bbox_4, t_angle_4, t_ctr_4,
           s_cls_0, s_bbox_0, s_angle_0, s_ctr_0,
           s_cls_1, s_bbox_1, s_angle_1, s_ctr_1,
           s_cls_2, s_bbox_2, s_angle_2, s_ctr_2,
           s_cls_3, s_bbox_3, s_angle_3, s_ctr_3,
           s_cls_4, s_bbox_4, s_angle_4, s_ctr_4):
    inp = dict(locals())
    args = []
    in_specs = []
    for pre in ('t', 's'):
        for nm in ('cls', 'bbox', 'angle', 'ctr'):
            for l in range(_NLVL):
                x = inp['%s_%s_%d' % (pre, nm, l)]
                args.append(x)
                in_specs.append(pl.BlockSpec(
                    (_GB,) + x.shape[1:], lambda g: (g, 0, 0, 0)))

    out = pl.pallas_call(
        _body,
        grid=(_B // _GB,),
        in_specs=in_specs,
        out_shape=jax.ShapeDtypeStruct((4,), jnp.float32),
        out_specs=pl.BlockSpec(memory_space=pltpu.SMEM),
        scratch_shapes=[pltpu.VMEM((_B * 64, 128), jnp.float32)] * 4
        + [pltpu.VMEM((_B * 64, 128), jnp.int32),
           pltpu.VMEM((_CLS, 64, 128), jnp.float32),
           pltpu.VMEM((_CLS, 64, 128), jnp.float32),
           pltpu.VMEM((6, 64, 128), jnp.float32),
           pltpu.VMEM((6, 64, 128), jnp.float32),
           pltpu.VMEM((64, 128), jnp.int32),
           pltpu.SMEM((1,), jnp.float32)],
    )(*args)
    return (out[0], out[1], out[2], out[3])


# speedup vs baseline: 1.1243x; 1.1243x over previous
import jax
import jax.numpy as jnp
from jax.experimental import pallas as pl
from jax.experimental.pallas import tpu as pltpu

_B = 16
_CLS = 16
_HS = (64, 32, 16, 8, 4)
_HH = tuple(h * h for h in _HS)
_NPB = sum(_HH)
_N = _B * _NPB
_K = max(int(_N * 0.01), 2)
_OFF = (0, 4096, 5120, 5376, 5440)
_NLVL = 5
_ONE_BITS = 0x3F800001
_LANE0 = (0, 64, 96, 112, 120)
_DEAD = 1 << 30
_GB = 2


def _smooth_l1(x, y):
    d = jnp.abs(x - y)
    return jnp.where(d < 1.0, 0.5 * d * d, d - 0.5)


def _rect(lvl):
    h = _HS[lvl]
    return slice(0, h), slice(_LANE0[lvl], _LANE0[lvl] + h)


def _pattern():
    row = jax.lax.broadcasted_iota(jnp.int32, (64, 128), 0)
    lane = jax.lax.broadcasted_iota(jnp.int32, (64, 128), 1)
    p = jnp.full((64, 128), _DEAD, jnp.int32)
    for l in range(_NLVL - 1, -1, -1):
        h = _HS[l]
        inside = (lane >= _LANE0[l]) & (lane < _LANE0[l] + h) & (row < h)
        p = jnp.where(inside, _OFF[l] + row * h + (lane - _LANE0[l]), p)
    return p


def _body(*refs):
    t_cls = refs[0:5]
    t_bbox = refs[5:10]
    t_angle = refs[10:15]
    t_ctr = refs[15:20]
    s_cls = refs[20:25]
    s_bbox = refs[25:30]
    s_angle = refs[30:35]
    s_ctr = refs[35:40]
    out = refs[40]
    f_s, f_dl, f_bb, f_ct, f_ix = refs[41:46]
    ts3, ss3, bt6, bs6 = refs[46:50]
    f_p, acc = refs[50:52]

    g = pl.program_id(0)

    @pl.when(g == 0)
    def _init():
        acc[0] = jnp.float32(0.0)
        f_p[...] = _pattern()
        ts3[...] = jnp.zeros(ts3.shape, jnp.float32)
        ss3[...] = jnp.zeros(ss3.shape, jnp.float32)
        bt6[...] = jnp.zeros(bt6.shape, jnp.float32)
        bs6[...] = jnp.zeros(bs6.shape, jnp.float32)

    valid = f_p[...] < _DEAD
    base_total = jnp.float32(0.0)
    for bb in range(_GB):
        b = g * _GB + bb
        blk = pl.ds(b * 64, 64)

        for l in range(_NLVL):
            r, c = _rect(l)
            ts3[:, r, c] = t_cls[l][bb]
            ss3[:, r, c] = s_cls[l][bb]
            bt6[0:4, r, c] = t_bbox[l][bb]
            bs6[0:4, r, c] = s_bbox[l][bb]
            bt6[4:5, r, c] = t_angle[l][bb]
            bs6[4:5, r, c] = s_angle[l][bb]
            bt6[5:6, r, c] = t_ctr[l][bb]
            bs6[5:6, r, c] = s_ctr[l][bb]

        x = ss3[...]
        ts = jax.nn.sigmoid(ts3[...])
        ps = jax.nn.sigmoid(x)
        s = jnp.max(ts, axis=0)
        f_s[blk] = jnp.where(valid, s, -1.0)
        f_ix[blk] = f_p[...] + b * _NPB

        l1p = jnp.log1p(-ps)
        base_total += jnp.sum(
            jnp.where(valid[None], l1p * jnp.square(ps), 0.0))
        d = ts - ps
        delta = ts * (x * jnp.square(d) + l1p * (d - ps))
        f_dl[blk] = -jnp.sum(delta, axis=0)

        bt = bt6[...]
        bs = bs6[...]
        d5 = jnp.sum(_smooth_l1(bs[0:5], bt[0:5]), axis=0)
        tc_ = jax.nn.sigmoid(bt[5])
        sc_ = jax.nn.sigmoid(bs[5])
        f_bb[blk] = d5 * tc_
        pcc = jnp.clip(sc_, 1e-12, 1.0 - 1e-12)
        f_ct[blk] = -(tc_ * jnp.log(pcc) + (1.0 - tc_) * jnp.log1p(-pcc))

    acc[0] += -base_total

    @pl.when(g == _B // _GB - 1)
    def _finish():
        _tail(out, f_s, f_dl, f_bb, f_ct, f_ix, acc)


def _tail(out, f_s, f_dl, f_bb, f_ct, f_ix, acc):
    base_total = acc[0]

    def bs_val(i, carry):
        lo, hi = carry
        mid = (lo + hi) // 2
        tf = jax.lax.bitcast_convert_type(mid, jnp.float32)
        big = jnp.sum((f_s[...] >= tf).astype(jnp.int32)) >= _K
        return (jnp.where(big, mid, lo), jnp.where(big, hi, mid))

    lo, _ = jax.lax.fori_loop(
        0, 31, bs_val, (jnp.int32(0), jnp.int32(_ONE_BITS)))
    T = jax.lax.bitcast_convert_type(lo, jnp.float32)

    s = f_s[...]
    ix = f_ix[...]
    c_gt = jnp.sum((s > T).astype(jnp.int32))
    c_tie = jnp.sum((s == T).astype(jnp.int32))
    m = _K - c_gt

    def bs_idx(i, carry):
        lo2, hi2 = carry
        mid = (lo2 + hi2) // 2
        c = jnp.sum(((f_s[...] == T) & (f_ix[...] <= mid)).astype(jnp.int32))
        ok = c >= m
        return (jnp.where(ok, lo2, mid), jnp.where(ok, mid, hi2))

    def _search_idx(_):
        return jax.lax.fori_loop(
            0, 17, bs_idx, (jnp.int32(-1), jnp.int32(_N - 1)))[1]

    I = jax.lax.cond(m == c_tie, lambda _: jnp.int32(_N), _search_idx, 0)

    mask = (s > T) | ((s == T) & (ix <= I))
    cls_sum = base_total + jnp.sum(jnp.where(mask, f_dl[...], 0.0))
    bbox_sum = jnp.sum(jnp.where(mask, f_bb[...], 0.0))
    ctr_sum = jnp.sum(jnp.where(mask, f_ct[...], 0.0))
    fg_num = jnp.sum(jnp.where(s > T, s, 0.0)) + m.astype(jnp.float32) * T
    s_sum = jnp.sum(jnp.maximum(s, 0.0))

    out[0] = cls_sum / fg_num
    out[1] = bbox_sum / jnp.float32(_K * 5)
    out[2] = ctr_sum / jnp.float32(_K)
    out[3] = s_sum / jnp.float32(_N)


def kernel(t_cls_0, t_bbox_0, t_angle_0, t_ctr_0,
           t_cls_1, t_bbox_1, t_angle_1, t_ctr_1,
           t_cls_2, t_bbox_2, t_angle_2, t_ctr_2,
           t_cls_3, t_bbox_3, t_angle_3, t_ctr_3,
           t_cls_4, t_bbox_4, t_angle_4, t_ctr_4,
           s_cls_0, s_bbox_0, s_angle_0, s_ctr_0,
           s_cls_1, s_bbox_1, s_angle_1, s_ctr_1,
           s_cls_2, s_bbox_2, s_angle_2, s_ctr_2,
           s_cls_3, s_bbox_3, s_angle_3, s_ctr_3,
           s_cls_4, s_bbox_4, s_angle_4, s_ctr_4):
    inp = dict(locals())
    args = []
    in_specs = []
    for pre in ('t', 's'):
        for nm in ('cls', 'bbox', 'angle', 'ctr'):
            for l in range(_NLVL):
                x = inp['%s_%s_%d' % (pre, nm, l)]
                args.append(x)
                in_specs.append(pl.BlockSpec(
                    (_GB,) + x.shape[1:], lambda g: (g, 0, 0, 0)))

    out = pl.pallas_call(
        _body,
        grid=(_B // _GB,),
        in_specs=in_specs,
        out_shape=jax.ShapeDtypeStruct((4,), jnp.float32),
        out_specs=pl.BlockSpec(memory_space=pltpu.SMEM),
        scratch_shapes=[pltpu.VMEM((_B * 64, 128), jnp.float32)] * 4
        + [pltpu.VMEM((_B * 64, 128), jnp.int32),
           pltpu.VMEM((_CLS, 64, 128), jnp.float32),
           pltpu.VMEM((_CLS, 64, 128), jnp.float32),
           pltpu.VMEM((6, 64, 128), jnp.float32),
           pltpu.VMEM((6, 64, 128), jnp.float32),
           pltpu.VMEM((64, 128), jnp.int32),
           pltpu.SMEM((1,), jnp.float32)],
    )(*args)
    return (out[0], out[1], out[2], out[3])


# speedup vs baseline: 1.1277x; 1.0030x over previous
import jax
import jax.numpy as jnp
from jax.experimental import pallas as pl
from jax.experimental.pallas import tpu as pltpu

_B = 16
_CLS = 16
_HS = (64, 32, 16, 8, 4)
_HH = tuple(h * h for h in _HS)
_NPB = sum(_HH)
_N = _B * _NPB
_K = max(int(_N * 0.01), 2)
_OFF = (0, 4096, 5120, 5376, 5440)
_NLVL = 5
_ONE_BITS = 0x3F800001
_LANE0 = (0, 64, 96, 112, 120)
_DEAD = 1 << 30
_GB = 2


def _sigmoid(x):
    return 1.0 / (1.0 + jnp.exp2(x * jnp.float32(-1.4426950408889634)))


def _smooth_l1(x, y):
    d = jnp.abs(x - y)
    return jnp.where(d < 1.0, 0.5 * d * d, d - 0.5)


def _rect(lvl):
    h = _HS[lvl]
    return slice(0, h), slice(_LANE0[lvl], _LANE0[lvl] + h)


def _pattern():
    row = jax.lax.broadcasted_iota(jnp.int32, (64, 128), 0)
    lane = jax.lax.broadcasted_iota(jnp.int32, (64, 128), 1)
    p = jnp.full((64, 128), _DEAD, jnp.int32)
    for l in range(_NLVL - 1, -1, -1):
        h = _HS[l]
        inside = (lane >= _LANE0[l]) & (lane < _LANE0[l] + h) & (row < h)
        p = jnp.where(inside, _OFF[l] + row * h + (lane - _LANE0[l]), p)
    return p


def _body(*refs):
    t_cls = refs[0:5]
    t_bbox = refs[5:10]
    t_angle = refs[10:15]
    t_ctr = refs[15:20]
    s_cls = refs[20:25]
    s_bbox = refs[25:30]
    s_angle = refs[30:35]
    s_ctr = refs[35:40]
    out = refs[40]
    f_s, f_dl, f_bb, f_ct, f_ix = refs[41:46]
    ts3, ss3, bt6, bs6 = refs[46:50]
    f_p, acc = refs[50:52]

    g = pl.program_id(0)

    @pl.when(g == 0)
    def _init():
        acc[0] = jnp.float32(0.0)
        f_p[...] = _pattern()
        ts3[...] = jnp.zeros(ts3.shape, jnp.float32)
        ss3[...] = jnp.zeros(ss3.shape, jnp.float32)
        bt6[...] = jnp.zeros(bt6.shape, jnp.float32)
        bs6[...] = jnp.zeros(bs6.shape, jnp.float32)

    valid = f_p[...] < _DEAD
    base_total = jnp.float32(0.0)
    for bb in range(_GB):
        b = g * _GB + bb
        blk = pl.ds(b * 64, 64)

        for l in range(_NLVL):
            r, c = _rect(l)
            ts3[:, r, c] = t_cls[l][bb]
            ss3[:, r, c] = s_cls[l][bb]
            bt6[0:4, r, c] = t_bbox[l][bb]
            bs6[0:4, r, c] = s_bbox[l][bb]
            bt6[4:5, r, c] = t_angle[l][bb]
            bs6[4:5, r, c] = s_angle[l][bb]
            bt6[5:6, r, c] = t_ctr[l][bb]
            bs6[5:6, r, c] = s_ctr[l][bb]

        x = ss3[...]
        ts = _sigmoid(ts3[...])
        ps = _sigmoid(x)
        s = jnp.max(ts, axis=0)
        f_s[blk] = jnp.where(valid, s, -1.0)
        f_ix[blk] = f_p[...] + b * _NPB

        l1p = jnp.log1p(-ps)
        base_total += jnp.sum(
            jnp.where(valid[None], l1p * jnp.square(ps), 0.0))
        d = ts - ps
        delta = ts * (x * jnp.square(d) + l1p * (d - ps))
        f_dl[blk] = -jnp.sum(delta, axis=0)

        bt = bt6[...]
        bs = bs6[...]
        d5 = jnp.sum(_smooth_l1(bs[0:5], bt[0:5]), axis=0)
        tc_ = _sigmoid(bt[5])
        sc_ = _sigmoid(bs[5])
        f_bb[blk] = d5 * tc_
        pcc = jnp.clip(sc_, 1e-12, 1.0 - 1e-12)
        f_ct[blk] = -(tc_ * jnp.log(pcc) + (1.0 - tc_) * jnp.log1p(-pcc))

    acc[0] += -base_total

    @pl.when(g == _B // _GB - 1)
    def _finish():
        _tail(out, f_s, f_dl, f_bb, f_ct, f_ix, acc)


def _tail(out, f_s, f_dl, f_bb, f_ct, f_ix, acc):
    base_total = acc[0]

    def bs_val(i, carry):
        lo, hi = carry
        mid = (lo + hi) // 2
        tf = jax.lax.bitcast_convert_type(mid, jnp.float32)
        big = jnp.sum((f_s[...] >= tf).astype(jnp.int32)) >= _K
        return (jnp.where(big, mid, lo), jnp.where(big, hi, mid))

    lo, _ = jax.lax.fori_loop(
        0, 30, bs_val, (jnp.int32(0), jnp.int32(_ONE_BITS)))
    T = jax.lax.bitcast_convert_type(lo, jnp.float32)

    s = f_s[...]
    ix = f_ix[...]
    c_gt = jnp.sum((s > T).astype(jnp.int32))
    c_tie = jnp.sum((s == T).astype(jnp.int32))
    m = _K - c_gt

    def bs_idx(i, carry):
        lo2, hi2 = carry
        mid = (lo2 + hi2) // 2
        c = jnp.sum(((f_s[...] == T) & (f_ix[...] <= mid)).astype(jnp.int32))
        ok = c >= m
        return (jnp.where(ok, lo2, mid), jnp.where(ok, mid, hi2))

    def _search_idx(_):
        return jax.lax.fori_loop(
            0, 17, bs_idx, (jnp.int32(-1), jnp.int32(_N - 1)))[1]

    I = jax.lax.cond(m == c_tie, lambda _: jnp.int32(_N), _search_idx, 0)

    mask = (s > T) | ((s == T) & (ix <= I))
    cls_sum = base_total + jnp.sum(jnp.where(mask, f_dl[...], 0.0))
    bbox_sum = jnp.sum(jnp.where(mask, f_bb[...], 0.0))
    ctr_sum = jnp.sum(jnp.where(mask, f_ct[...], 0.0))
    fg_num = jnp.sum(jnp.where(s > T, s, 0.0)) + m.astype(jnp.float32) * T
    s_sum = jnp.sum(jnp.maximum(s, 0.0))

    out[0] = cls_sum / fg_num
    out[1] = bbox_sum / jnp.float32(_K * 5)
    out[2] = ctr_sum / jnp.float32(_K)
    out[3] = s_sum / jnp.float32(_N)


def kernel(t_cls_0, t_bbox_0, t_angle_0, t_ctr_0,
           t_cls_1, t_bbox_1, t_angle_1, t_ctr_1,
           t_cls_2, t_bbox_2, t_angle_2, t_ctr_2,
           t_cls_3, t_bbox_3, t_angle_3, t_ctr_3,
           t_cls_4, t_bbox_4, t_angle_4, t_ctr_4,
           s_cls_0, s_bbox_0, s_angle_0, s_ctr_0,
           s_cls_1, s_bbox_1, s_angle_1, s_ctr_1,
           s_cls_2, s_bbox_2, s_angle_2, s_ctr_2,
           s_cls_3, s_bbox_3, s_angle_3, s_ctr_3,
           s_cls_4, s_bbox_4, s_angle_4, s_ctr_4):
    inp = dict(locals())
    args = []
    in_specs = []
    for pre in ('t', 's'):
        for nm in ('cls', 'bbox', 'angle', 'ctr'):
            for l in range(_NLVL):
                x = inp['%s_%s_%d' % (pre, nm, l)]
                args.append(x)
                in_specs.append(pl.BlockSpec(
                    (_GB,) + x.shape[1:], lambda g: (g, 0, 0, 0)))

    out = pl.pallas_call(
        _body,
        grid=(_B // _GB,),
        in_specs=in_specs,
        out_shape=jax.ShapeDtypeStruct((4,), jnp.float32),
        out_specs=pl.BlockSpec(memory_space=pltpu.SMEM),
        scratch_shapes=[pltpu.VMEM((_B * 64, 128), jnp.float32)] * 4
        + [pltpu.VMEM((_B * 64, 128), jnp.int32),
           pltpu.VMEM((_CLS, 64, 128), jnp.float32),
           pltpu.VMEM((_CLS, 64, 128), jnp.float32),
           pltpu.VMEM((6, 64, 128), jnp.float32),
           pltpu.VMEM((6, 64, 128), jnp.float32),
           pltpu.VMEM((64, 128), jnp.int32),
           pltpu.SMEM((1,), jnp.float32)],
    )(*args)
    return (out[0], out[1], out[2], out[3])
